# Initial kernel scaffold; baseline (speedup 1.0000x reference)
import jax
import jax.numpy as jnp
from jax.experimental import pallas as pl


def kernel(preds, nearest_gt, gt_normals, edge_list):
    raise NotImplementedError("write your pallas kernel here")


# speedup vs baseline: 107.2614x; 107.2614x over previous
import jax
import jax.numpy as jnp
from jax import lax
from jax.experimental import pallas as pl
from jax.experimental.pallas import tpu as pltpu
from jax.experimental.pallas import tpu_sc as plsc

_NC = 2
_NS = 16
_NW = _NC * _NS
_L = 16


def _build(B, N, E):
    TOT = B * E
    assert TOT % _NW == 0
    EPW = TOT // _NW
    assert E % EPW == 0
    WPB = E // EPW
    K = 2000
    assert EPW % K == 0 and K % _L == 0 and K % 8 == 0
    NCHUNK = EPW // K

    mesh = plsc.VectorSubcoreMesh(core_axis_name="c", subcore_axis_name="s")

    def body(i0_hbm, i1_hbm, ng_hbm, px_hbm, py_hbm, pz_hbm,
             nx_hbm, ny_hbm, nz_hbm, out_hbm,
             i0_v, i1_v, g_v,
             p0x_v, p0y_v, p0z_v, p1x_v, p1y_v, p1z_v,
             nx_v, ny_v, nz_v, st_v,
             sem_g, sem_p, sem_n):
        c = lax.axis_index("c")
        s = lax.axis_index("s")
        wid = s * _NC + c
        bN = (wid // WPB) * N

        eps2 = jnp.float32(1e-24)
        one = jnp.float32(1.0)
        zero = jnp.float32(0.0)
        z16 = jnp.zeros((_L,), jnp.float32)

        def chunk_body(ci, carry):
            sacc0, cacc0 = carry
            base = wid * EPW + ci * K
            pltpu.sync_copy(i0_hbm.at[pl.ds(base, K)], i0_v)
            pltpu.sync_copy(i1_hbm.at[pl.ds(base, K)], i1_v)
            cg = pltpu.async_copy(ng_hbm.at[i0_v], g_v, sem_g)
            cps = [
                pltpu.async_copy(px_hbm.at[i0_v], p0x_v, sem_p),
                pltpu.async_copy(py_hbm.at[i0_v], p0y_v, sem_p),
                pltpu.async_copy(pz_hbm.at[i0_v], p0z_v, sem_p),
                pltpu.async_copy(px_hbm.at[i1_v], p1x_v, sem_p),
                pltpu.async_copy(py_hbm.at[i1_v], p1y_v, sem_p),
                pltpu.async_copy(pz_hbm.at[i1_v], p1z_v, sem_p),
            ]
            cg.wait()
            cns = [
                pltpu.async_copy(nx_hbm.at[g_v], nx_v, sem_n),
                pltpu.async_copy(ny_hbm.at[g_v], ny_v, sem_n),
                pltpu.async_copy(nz_hbm.at[g_v], nz_v, sem_n),
            ]
            for cp in cps:
                cp.wait()
            for cn in cns:
                cn.wait()

            def vec_body(vi, carry2):
                sa, ca = carry2
                sl = pl.ds(vi * _L, _L)
                i0x = i0_v[sl]
                i1x = i1_v[sl]
                m = jnp.where((i0x != bN) | (i1x != bN), one, zero)
                dx = p0x_v[sl] - p1x_v[sl]
                dy = p0y_v[sl] - p1y_v[sl]
                dz = p0z_v[sl] - p1z_v[sl]
                nx = nx_v[sl]
                ny = ny_v[sl]
                nz = nz_v[sl]
                dn = dx * nx + dy * ny + dz * nz
                dd = dx * dx + dy * dy + dz * dz
                nn = nx * nx + ny * ny + nz * nz
                u = (dn * dn) / jnp.maximum(dd, eps2)
                l = u / jnp.maximum(nn, eps2)
                return (sa + l * m, ca + m)

            return lax.fori_loop(0, K // _L, vec_body, (sacc0, cacc0))

        sacc, cacc = lax.fori_loop(0, NCHUNK, chunk_body, (z16, z16))
        st_v[pl.ds(0, _L)] = sacc
        st_v[pl.ds(_L, _L)] = cacc
        pltpu.sync_copy(st_v, out_hbm.at[wid])

    fvec = pltpu.VMEM((K,), jnp.float32)
    ivec = pltpu.VMEM((K,), jnp.int32)
    return pl.kernel(
        body,
        out_type=jax.ShapeDtypeStruct((_NW, 2 * _L), jnp.float32),
        mesh=mesh,
        scratch_types=[
            ivec, ivec, ivec,
            fvec, fvec, fvec, fvec, fvec, fvec,
            fvec, fvec, fvec,
            pltpu.VMEM((2 * _L,), jnp.float32),
            pltpu.SemaphoreType.DMA,
            pltpu.SemaphoreType.DMA,
            pltpu.SemaphoreType.DMA,
        ],
    )


def kernel(preds, nearest_gt, gt_normals, edge_list):
    B, N, _ = preds.shape
    E = edge_list.shape[2]
    offs = (jnp.arange(B, dtype=jnp.int32) * N)[:, None]
    i0 = (edge_list[:, 0, :] + offs).reshape(-1)
    i1 = (edge_list[:, 1, :] + offs).reshape(-1)
    ng = (nearest_gt + offs).reshape(-1)
    px, py, pz = [preds[:, :, d].reshape(-1) for d in range(3)]
    nx, ny, nz = [gt_normals[:, :, d].reshape(-1) for d in range(3)]

    out = _build(B, N, E)(i0, i1, ng, px, py, pz, nx, ny, nz)
    loss_sum = jnp.sum(out[:, :_L])
    cnt = jnp.sum(out[:, _L:])
    return loss_sum / jnp.maximum(cnt, 1.0)
